# speedup vs baseline: 1.4903x; 1.4903x over previous
import functools

import jax
import jax.numpy as jnp
import numpy as np
from jax.experimental import pallas as pl
from jax.experimental.pallas import tpu as pltpu

B, S, D = 2, 2048, 768
E, F = 64, 48
EF = E * F
TOP_K = 2
N = B * S

_TBLK = 512


def _moe_dense_body(x_ref, wrt_ref, w1t_ref, b1_ref, w2f_ref, b2_ref,
                    expand_ref, out_ref):
    xb = x_ref[...]
    logits = jnp.dot(xb, wrt_ref[...],
                     preferred_element_type=jnp.float32)
    T = xb.shape[0]
    iota = jax.lax.broadcasted_iota(jnp.int32, (T, E), 1)
    m1 = jnp.max(logits, axis=1, keepdims=True)
    idx1 = jnp.min(jnp.where(logits == m1, iota, E), axis=1, keepdims=True)
    sel1 = iota == idx1
    logits2 = jnp.where(sel1, -jnp.inf, logits)
    m2 = jnp.max(logits2, axis=1, keepdims=True)
    idx2 = jnp.min(jnp.where(logits2 == m2, iota, E), axis=1, keepdims=True)
    sel2 = iota == idx2
    e2 = jnp.exp(m2 - m1)
    g1 = 1.0 / (1.0 + e2)
    g2 = e2 / (1.0 + e2)
    gates = jnp.where(sel1, g1, 0.0) + jnp.where(sel2, g2, 0.0)

    h = jnp.dot(xb, w1t_ref[...], preferred_element_type=jnp.float32)
    h = h + b1_ref[...]
    h = jax.nn.gelu(h, approximate=True)
    gates_e = jnp.dot(gates, expand_ref[...],
                      preferred_element_type=jnp.float32)
    hg = h * gates_e
    out = jnp.dot(hg, w2f_ref[...], preferred_element_type=jnp.float32)
    out = out + jnp.dot(gates, b2_ref[...], preferred_element_type=jnp.float32)
    out_ref[...] = out


@functools.partial(jax.jit, static_argnames=("interpret",))
def _moe_dense(x, Wr, W1, b1, W2, b2, interpret=False):
    flat_x = x.reshape(N, D)
    wrt = Wr.T
    w1t = W1.reshape(EF, D).T
    b1f = b1.reshape(1, EF)
    w2f = W2.transpose(0, 2, 1).reshape(EF, D)
    expand = jnp.asarray(np.kron(np.eye(E, dtype=np.float32),
                                 np.ones((1, F), dtype=np.float32)))

    grid = (N // _TBLK,)
    out = pl.pallas_call(
        _moe_dense_body,
        grid=grid,
        in_specs=[
            pl.BlockSpec((_TBLK, D), lambda i: (i, 0)),
            pl.BlockSpec((D, E), lambda i: (0, 0)),
            pl.BlockSpec((D, EF), lambda i: (0, 0)),
            pl.BlockSpec((1, EF), lambda i: (0, 0)),
            pl.BlockSpec((EF, D), lambda i: (0, 0)),
            pl.BlockSpec((E, D), lambda i: (0, 0)),
            pl.BlockSpec((E, EF), lambda i: (0, 0)),
        ],
        out_specs=pl.BlockSpec((_TBLK, D), lambda i: (i, 0)),
        out_shape=jax.ShapeDtypeStruct((N, D), jnp.float32),
        interpret=interpret,
    )(flat_x, wrt, w1t, b1f, w2f, b2, expand)
    return out.reshape(B, S, D)


def kernel(x, Wr, W1, b1, W2, b2):
    return _moe_dense(x, Wr, W1, b1, W2, b2)


# speedup vs baseline: 1.5839x; 1.0629x over previous
import functools

import jax
import jax.numpy as jnp
import numpy as np
from jax.experimental import pallas as pl
from jax.experimental.pallas import tpu as pltpu

B, S, D = 2, 2048, 768
E, F = 64, 48
EF = E * F
TOP_K = 2
N = B * S

_TBLK = 512


def _moe_dense_body(x_ref, xbf_ref, wrt_ref, w1t_ref, b1_ref, w2f_ref, b2_ref,
                    expand_ref, out_ref):
    xb = x_ref[...]
    xbf = xbf_ref[...]
    logits = jnp.dot(xb, wrt_ref[...],
                     preferred_element_type=jnp.float32)
    T = xb.shape[0]
    iota = jax.lax.broadcasted_iota(jnp.int32, (T, E), 1)
    m1 = jnp.max(logits, axis=1, keepdims=True)
    idx1 = jnp.min(jnp.where(logits == m1, iota, E), axis=1, keepdims=True)
    sel1 = iota == idx1
    logits2 = jnp.where(sel1, -jnp.inf, logits)
    m2 = jnp.max(logits2, axis=1, keepdims=True)
    idx2 = jnp.min(jnp.where(logits2 == m2, iota, E), axis=1, keepdims=True)
    sel2 = iota == idx2
    e2 = jnp.exp(m2 - m1)
    g1 = 1.0 / (1.0 + e2)
    g2 = e2 / (1.0 + e2)
    gates = jnp.where(sel1, g1, 0.0) + jnp.where(sel2, g2, 0.0)

    h = jnp.dot(xbf, w1t_ref[...], preferred_element_type=jnp.float32)
    h = h + b1_ref[...]
    h = jax.nn.gelu(h, approximate=True)
    gates_e = jnp.dot(gates, expand_ref[...],
                      preferred_element_type=jnp.float32)
    hg = (h * gates_e).astype(jnp.bfloat16)
    out = jnp.dot(hg, w2f_ref[...], preferred_element_type=jnp.float32)
    out = out + jnp.dot(gates, b2_ref[...], preferred_element_type=jnp.float32)
    out_ref[...] = out


@functools.partial(jax.jit, static_argnames=("interpret",))
def _moe_dense(x, Wr, W1, b1, W2, b2, interpret=False):
    flat_x = x.reshape(N, D)
    xbf = flat_x.astype(jnp.bfloat16)
    wrt = Wr.T
    w1t = W1.reshape(EF, D).T.astype(jnp.bfloat16)
    b1f = b1.reshape(1, EF)
    w2f = W2.transpose(0, 2, 1).reshape(EF, D).astype(jnp.bfloat16)
    expand = jnp.asarray(np.kron(np.eye(E, dtype=np.float32),
                                 np.ones((1, F), dtype=np.float32)))

    grid = (N // _TBLK,)
    out = pl.pallas_call(
        _moe_dense_body,
        grid=grid,
        in_specs=[
            pl.BlockSpec((_TBLK, D), lambda i: (i, 0)),
            pl.BlockSpec((_TBLK, D), lambda i: (i, 0)),
            pl.BlockSpec((D, E), lambda i: (0, 0)),
            pl.BlockSpec((D, EF), lambda i: (0, 0)),
            pl.BlockSpec((1, EF), lambda i: (0, 0)),
            pl.BlockSpec((EF, D), lambda i: (0, 0)),
            pl.BlockSpec((E, D), lambda i: (0, 0)),
            pl.BlockSpec((E, EF), lambda i: (0, 0)),
        ],
        out_specs=pl.BlockSpec((_TBLK, D), lambda i: (i, 0)),
        out_shape=jax.ShapeDtypeStruct((N, D), jnp.float32),
        interpret=interpret,
    )(flat_x, xbf, wrt, w1t, b1f, w2f, b2, expand)
    return out.reshape(B, S, D)


def kernel(x, Wr, W1, b1, W2, b2):
    return _moe_dense(x, Wr, W1, b1, W2, b2)


# speedup vs baseline: 1.7815x; 1.1247x over previous
import functools

import jax
import jax.numpy as jnp
import numpy as np
from jax.experimental import pallas as pl
from jax.experimental.pallas import tpu as pltpu

B, S, D = 2, 2048, 768
E, F = 64, 48
EF = E * F
TOP_K = 2
N = B * S

_TBLK = 512
_NCHUNK = 6


def _moe_dense_body(x_ref, wrt_ref, w1f_ref, b1_ref, w2f_ref, b2_ref,
                    expand_ref, out_ref):
    xb = x_ref[...]
    xbf = xb.astype(jnp.bfloat16)
    logits = jnp.dot(xb, wrt_ref[...],
                     preferred_element_type=jnp.float32)
    T = xb.shape[0]
    iota = jax.lax.broadcasted_iota(jnp.int32, (T, E), 1)
    m1 = jnp.max(logits, axis=1, keepdims=True)
    idx1 = jnp.min(jnp.where(logits == m1, iota, E), axis=1, keepdims=True)
    sel1 = iota == idx1
    logits2 = jnp.where(sel1, -jnp.inf, logits)
    m2 = jnp.max(logits2, axis=1, keepdims=True)
    idx2 = jnp.min(jnp.where(logits2 == m2, iota, E), axis=1, keepdims=True)
    sel2 = iota == idx2
    e2 = jnp.exp(m2 - m1)
    g1 = 1.0 / (1.0 + e2)
    g2 = e2 / (1.0 + e2)
    gates = jnp.where(sel1, g1, 0.0) + jnp.where(sel2, g2, 0.0)

    gates_e = jnp.dot(gates.astype(jnp.bfloat16), expand_ref[...],
                      preferred_element_type=jnp.float32
                      ).astype(jnp.bfloat16)
    c0 = jnp.bfloat16(0.7978845608028654)
    c1 = jnp.bfloat16(0.044715)
    h = jax.lax.dot_general(xbf, w1f_ref[...], (((1,), (1,)), ((), ())),
                            preferred_element_type=jnp.float32)
    h = (h + b1_ref[...]).astype(jnp.bfloat16)
    t = jnp.tanh(c0 * (h + c1 * h * h * h))
    hg = (h * (jnp.bfloat16(1.0) + t)) * gates_e
    out = jnp.dot(hg, w2f_ref[...], preferred_element_type=jnp.float32)
    out = out + jnp.dot(gates, b2_ref[...], preferred_element_type=jnp.float32)
    out_ref[...] = out


@functools.partial(jax.jit, static_argnames=("interpret",))
def _moe_dense(x, Wr, W1, b1, W2, b2, interpret=False):
    flat_x = x.reshape(N, D)
    wrt = Wr.T
    w1f = W1.reshape(EF, D).astype(jnp.bfloat16)
    b1f = b1.reshape(1, EF)
    w2f = W2.astype(jnp.bfloat16).transpose(0, 2, 1).reshape(EF, D)
    expand = jnp.asarray(np.kron(np.eye(E, dtype=np.float32),
                                 np.full((1, F), 0.5, dtype=np.float32))
                         ).astype(jnp.bfloat16)

    grid = (N // _TBLK,)
    out = pl.pallas_call(
        _moe_dense_body,
        grid=grid,
        in_specs=[
            pl.BlockSpec((_TBLK, D), lambda i: (i, 0)),
            pl.BlockSpec((D, E), lambda i: (0, 0)),
            pl.BlockSpec((EF, D), lambda i: (0, 0)),
            pl.BlockSpec((1, EF), lambda i: (0, 0)),
            pl.BlockSpec((EF, D), lambda i: (0, 0)),
            pl.BlockSpec((E, D), lambda i: (0, 0)),
            pl.BlockSpec((E, EF), lambda i: (0, 0)),
        ],
        out_specs=pl.BlockSpec((_TBLK, D), lambda i: (i, 0)),
        out_shape=jax.ShapeDtypeStruct((N, D), jnp.float32),
        interpret=interpret,
    )(flat_x, wrt, w1f, b1f, w2f, b2, expand)
    return out.reshape(B, S, D)


def kernel(x, Wr, W1, b1, W2, b2):
    return _moe_dense(x, Wr, W1, b1, W2, b2)


# speedup vs baseline: 1.9195x; 1.0775x over previous
import functools

import jax
import jax.numpy as jnp
import numpy as np
from jax.experimental import pallas as pl
from jax.experimental.pallas import tpu as pltpu

B, S, D = 2, 2048, 768
E, F = 64, 48
EF = E * F
TOP_K = 2
N = B * S

_TBLK = 1024
_NHALF = 4


def _moe_dense_body(x_ref, wr_ref, w1f_ref, b1_ref, w2f_ref, b2_ref,
                    expand_ref, out_ref, w1bf_ref):
    @pl.when(pl.program_id(0) == 0)
    def _prep():
        w1bf_ref[...] = w1f_ref[...].astype(jnp.bfloat16)

    xb = x_ref[...]
    xbf = xb.astype(jnp.bfloat16)
    logits = jax.lax.dot_general(xb, wr_ref[...], (((1,), (1,)), ((), ())),
                                 preferred_element_type=jnp.float32)
    T = xb.shape[0]
    iota = jax.lax.broadcasted_iota(jnp.int32, (T, E), 1)
    m1 = jnp.max(logits, axis=1, keepdims=True)
    idx1 = jnp.min(jnp.where(logits == m1, iota, E), axis=1, keepdims=True)
    sel1 = iota == idx1
    logits2 = jnp.where(sel1, -jnp.inf, logits)
    m2 = jnp.max(logits2, axis=1, keepdims=True)
    idx2 = jnp.min(jnp.where(logits2 == m2, iota, E), axis=1, keepdims=True)
    sel2 = iota == idx2
    e2 = jnp.exp(m2 - m1)
    g1 = 1.0 / (1.0 + e2)
    g2 = e2 / (1.0 + e2)
    gates = jnp.where(sel1, g1, 0.0) + jnp.where(sel2, g2, 0.0)
    gates_bf = gates.astype(jnp.bfloat16)

    gates_e = jnp.dot(gates_bf, expand_ref[...],
                      preferred_element_type=jnp.float32
                      ).astype(jnp.bfloat16)
    c0 = jnp.bfloat16(0.7978845608028654)
    c1 = jnp.bfloat16(0.044715)
    bias = jnp.dot(gates_bf, b2_ref[...], preferred_element_type=jnp.float32)
    H = T // _NHALF
    for r in range(_NHALF):
        rs = slice(r * H, (r + 1) * H)
        h = jax.lax.dot_general(xbf[rs], w1bf_ref[...],
                                (((1,), (1,)), ((), ())),
                                preferred_element_type=jnp.float32)
        h = (h + b1_ref[...]).astype(jnp.bfloat16)
        h2 = h * h
        t = jnp.tanh(h * (c0 + (c0 * c1) * h2))
        hg = (h + h * t) * gates_e[rs]
        out = jnp.dot(hg, w2f_ref[...], preferred_element_type=jnp.float32)
        out_ref[rs, :] = out + bias[rs]


@functools.partial(jax.jit, static_argnames=("interpret",))
def _moe_dense(x, Wr, W1, b1, W2, b2, interpret=False):
    flat_x = x.reshape(N, D)
    w1f = W1.reshape(EF, D)
    b1f = b1.reshape(1, EF)
    w2f = W2.astype(jnp.bfloat16).transpose(0, 2, 1).reshape(EF, D)
    b2bf = b2.astype(jnp.bfloat16)
    expand = jnp.asarray(np.kron(np.eye(E, dtype=np.float32),
                                 np.full((1, F), 0.5, dtype=np.float32))
                         ).astype(jnp.bfloat16)

    grid = (N // _TBLK,)
    out = pl.pallas_call(
        _moe_dense_body,
        grid=grid,
        in_specs=[
            pl.BlockSpec((_TBLK, D), lambda i: (i, 0)),
            pl.BlockSpec((E, D), lambda i: (0, 0)),
            pl.BlockSpec((EF, D), lambda i: (0, 0)),
            pl.BlockSpec((1, EF), lambda i: (0, 0)),
            pl.BlockSpec((EF, D), lambda i: (0, 0)),
            pl.BlockSpec((E, D), lambda i: (0, 0)),
            pl.BlockSpec((E, EF), lambda i: (0, 0)),
        ],
        out_specs=pl.BlockSpec((_TBLK, D), lambda i: (i, 0)),
        out_shape=jax.ShapeDtypeStruct((N, D), jnp.float32),
        scratch_shapes=[pltpu.VMEM((EF, D), jnp.bfloat16)],
        interpret=interpret,
    )(flat_x, Wr, w1f, b1f, w2f, b2bf, expand)
    return out.reshape(B, S, D)


def kernel(x, Wr, W1, b1, W2, b2):
    return _moe_dense(x, Wr, W1, b1, W2, b2)
